# baseline (speedup 1.0000x reference)
import jax
import jax.numpy as jnp
from jax import lax
from jax.experimental import pallas as pl
from jax.experimental.pallas import tpu as pltpu
from jax.experimental.pallas import tpu_sc as plsc

B = 16384
D = 64
NC = 2
NS = 16
NW = NC * NS
BPW = B // NW
CHUNK = 64
NCHUNK = BPW // CHUNK

V_USER = 1000000
V_ITEM = 100000
V_CAT = 1000
V_BRAND = 1000

_PARAMS = pltpu.CompilerParams(use_tc_tiling_on_sc=True,
                               needs_layout_passes=False)
_MESH = plsc.VectorSubcoreMesh(core_axis_name="c", subcore_axis_name="s")


def _gat_body(u_id, i_id, c_id, b_id, us, is_, cs, bs, out_hbm,
              idx_raw, gidx_v, pb_v, bufs, obuf, gsem, wsem):
    wid = lax.axis_index("s") * NC + lax.axis_index("c")
    base = wid * BPW
    ids_hbm = (u_id, i_id, c_id, b_id)
    tabs = (us, is_, cs, bs)

    for f in range(4):
        pltpu.sync_copy(ids_hbm[f].at[pl.ds(base, BPW)], idx_raw.at[f])
    for f in range(4):
        for i in range(BPW // 16):
            v = idx_raw[f, pl.ds(i * 16, 16)]
            ci, lane = (i * 16) // CHUNK, (i * 16) % CHUNK
            gidx_v[f, ci, pl.ds(lane, 16)] = lax.shift_right_logical(v, 1)
            pb_v[f, ci, pl.ds(lane, 16)] = lax.shift_left(
                lax.bitwise_and(v, jnp.int32(1)), 6)

    def fire(ci, s):
        return [
            pltpu.async_copy(
                tabs[f].at[gidx_v.at[f, ci]], bufs.at[s, f], gsem)
            for f in range(4)
        ]

    def extract(ci, s):
        for f in range(4):
            for g in range(CHUNK // 16):
                jv = lax.iota(jnp.int32, 16) + g * 16
                pb = pb_v[f, ci, pl.ds(g * 16, 16)]
                orow = 2 * jv + (1 if f >= 2 else 0)
                ocb = (f % 2) * 64

                def wloop(w, _):
                    wv = jnp.full((16,), w, dtype=jnp.int32)
                    vals = plsc.load_gather(
                        bufs, [jnp.full((16,), s, jnp.int32),
                               jnp.full((16,), f, jnp.int32),
                               jv, pb + wv])
                    plsc.store_scatter(
                        obuf, [jnp.full((16,), s, jnp.int32),
                               orow, wv + ocb], vals)
                    return _
                lax.fori_loop(0, D, wloop, 0)

    g = [None] * NCHUNK
    w = [None] * NCHUNK
    g[0] = fire(0, 0)
    for ci in range(NCHUNK):
        s = ci % 2
        if ci + 1 < NCHUNK:
            g[ci + 1] = fire(ci + 1, 1 - s)
        for c in g[ci]:
            c.wait()
        if ci >= 2:
            w[ci - 2].wait()
        extract(ci, s)
        cbase = (base + ci * CHUNK) * 2
        w[ci] = pltpu.async_copy(
            obuf.at[s], out_hbm.at[pl.ds(cbase, 2 * CHUNK)], wsem)
    w[NCHUNK - 2].wait()
    w[NCHUNK - 1].wait()


_gat_call = pl.kernel(
    _gat_body,
    out_type=jax.ShapeDtypeStruct((2 * B, 128), jnp.float32),
    mesh=_MESH,
    compiler_params=_PARAMS,
    scratch_types=[
        pltpu.VMEM((4, BPW), jnp.int32),
        pltpu.VMEM((4, NCHUNK, CHUNK), jnp.int32),
        pltpu.VMEM((4, NCHUNK, CHUNK), jnp.int32),
        pltpu.VMEM((2, 4, CHUNK, 128), jnp.float32),
        pltpu.VMEM((2, 2 * CHUNK, 128), jnp.float32),
        pltpu.SemaphoreType.DMA,
        pltpu.SemaphoreType.DMA,
    ],
)


def kernel(user_id, item_id, category, brand,
           user_table, item_table, category_table, brand_table):
    us = user_table.reshape(V_USER // 2, 128)
    is_ = item_table.reshape(V_ITEM // 2, 128)
    cs = category_table.reshape(V_CAT // 2, 128)
    bs = brand_table.reshape(V_BRAND // 2, 128)
    out = _gat_call(user_id, item_id, category, brand, us, is_, cs, bs)
    return out.reshape(B, 4 * D)
